# Initial kernel scaffold; baseline (speedup 1.0000x reference)
import jax
import jax.numpy as jnp
from jax.experimental import pallas as pl


def kernel(doc_w, doc_c, qry_w, qry_c, k_layer, K, W, char_table, conv_w, conv_b):
    raise NotImplementedError("write your pallas kernel here")


# speedup vs baseline: 3.7137x; 3.7137x over previous
import functools

import jax
import jax.numpy as jnp
from jax import lax
from jax.experimental import pallas as pl
from jax.experimental.pallas import tpu as pltpu
from jax.experimental.pallas import tpu_sc as plsc

VOCAB = 100000
EMB = 128
NCHAR = 128
CDIM = 16
FSIZE = 64
FWIDTH = 5
B = 64
DL = 512
QL = 32
WL = 16

NW_TOTAL = B * DL + B * QL


_NC = 2
_NS = 16
_NWK = _NC * _NS
_PER_W = NW_TOTAL // _NWK
_NCHUNK = 17
_CHUNK = _PER_W // _NCHUNK


def _sc_gather_body(tbl_hbm, idx_hbm, out_hbm, idx_v, rows_v, sem):
    wid = lax.axis_index("s") * _NC + lax.axis_index("c")
    pltpu.sync_copy(idx_hbm.at[wid], idx_v)

    def step(c, _):
        pltpu.async_copy(tbl_hbm.at[idx_v.at[c]], rows_v, sem).wait()
        base = wid * _PER_W + c * _CHUNK
        pltpu.sync_copy(rows_v, out_hbm.at[pl.ds(base, _CHUNK)])
        return ()

    lax.fori_loop(0, _NCHUNK, step, (), unroll=False)


@jax.jit
def _sc_gather(table, idx):
    kern = pl.kernel(
        _sc_gather_body,
        out_type=jax.ShapeDtypeStruct((NW_TOTAL, EMB), jnp.float32),
        mesh=plsc.VectorSubcoreMesh(core_axis_name="c", subcore_axis_name="s"),
        scratch_types=[
            pltpu.VMEM((_NCHUNK, _CHUNK), jnp.int32),
            pltpu.VMEM((_CHUNK, EMB), jnp.float32),
            pltpu.SemaphoreType.DMA,
        ],
    )
    return kern(table, idx.reshape(_NWK, _NCHUNK, _CHUNK))


_BW = 256
_NB = _BW * WL
_GRID = NW_TOTAL // _BW


def _conv_body(ids_ref, tbl_ref, w80_ref, b_ref, out_ref):
    ids = ids_ref[...]
    onehot = (ids[:, :, None] == lax.broadcasted_iota(
        jnp.int32, (1, 1, NCHAR), 2)).astype(jnp.float32)
    onehot = onehot.reshape(_NB, NCHAR)
    e = jnp.dot(onehot, tbl_ref[...], preferred_element_type=jnp.float32)
    e = jnp.concatenate([e, jnp.zeros((FWIDTH - 1, CDIM), jnp.float32)], axis=0)
    x = jnp.concatenate([e[d:d + _NB, :] for d in range(FWIDTH)], axis=1)
    y = jnp.dot(x, w80_ref[...], preferred_element_type=jnp.float32)
    y = jnp.maximum(y + b_ref[...], 0.0)
    pos = lax.broadcasted_iota(jnp.int32, (_NB, 1), 0) % WL
    y = jnp.where(pos < WL - FWIDTH + 1, y, 0.0)
    out_ref[...] = jnp.max(y.reshape(_BW, WL, FSIZE), axis=1)


@jax.jit
def _tc_charconv(cidx, char_table, w80, b_row):
    return pl.pallas_call(
        _conv_body,
        out_shape=jax.ShapeDtypeStruct((NW_TOTAL, FSIZE), jnp.float32),
        grid=(_GRID,),
        in_specs=[
            pl.BlockSpec((_BW, WL), lambda i: (i, 0)),
            pl.BlockSpec((NCHAR, CDIM), lambda i: (0, 0)),
            pl.BlockSpec((FWIDTH * CDIM, FSIZE), lambda i: (0, 0)),
            pl.BlockSpec((1, FSIZE), lambda i: (0, 0)),
        ],
        out_specs=pl.BlockSpec((_BW, FSIZE), lambda i: (i, 0)),
    )(cidx, char_table, w80, b_row)


def kernel(doc_w, doc_c, qry_w, qry_c, k_layer, K, W, char_table, conv_w, conv_b):
    widx = jnp.concatenate(
        [doc_w.reshape(-1), qry_w.reshape(-1)]).astype(jnp.int32)
    cidx = jnp.concatenate(
        [doc_c.reshape(-1, WL), qry_c.reshape(-1, WL)]).astype(jnp.int32)

    w80 = jnp.transpose(conv_w[:, :, 0, :], (2, 1, 0)).reshape(
        FWIDTH * CDIM, FSIZE)
    b_row = conv_b.reshape(1, FSIZE)

    w_emb = _sc_gather(W, widx)
    c_emb = _tc_charconv(cidx, char_table, w80, b_row)

    full = jnp.concatenate([w_emb, c_emb], axis=1)
    doc_emb = full[:B * DL].reshape(B, DL, EMB + FSIZE)
    qry_emb = full[B * DL:].reshape(B, QL, EMB + FSIZE)
    return doc_emb, qry_emb


# speedup vs baseline: 7.6341x; 2.0556x over previous
import functools

import jax
import jax.numpy as jnp
from jax import lax
from jax.experimental import pallas as pl
from jax.experimental.pallas import tpu as pltpu
from jax.experimental.pallas import tpu_sc as plsc

VOCAB = 100000
EMB = 128
NCHAR = 128
CDIM = 16
FSIZE = 64
FWIDTH = 5
B = 64
DL = 512
QL = 32
WL = 16

NW_TOTAL = B * DL + B * QL


_NC = 2
_NS = 16
_NWK = _NC * _NS
_PER_W = NW_TOTAL // _NWK
_NCHUNK = 17
_CHUNK = _PER_W // _NCHUNK


def _sc_gather_body(tbl_hbm, idx_hbm, out_hbm, idx_v, rows_v, sem):
    wid = lax.axis_index("s") * _NC + lax.axis_index("c")
    pltpu.sync_copy(idx_hbm.at[wid], idx_v)

    def step(c, _):
        pltpu.async_copy(tbl_hbm.at[idx_v.at[c]], rows_v, sem).wait()
        base = wid * _PER_W + c * _CHUNK
        pltpu.sync_copy(rows_v, out_hbm.at[pl.ds(base, _CHUNK)])
        return ()

    lax.fori_loop(0, _NCHUNK, step, (), unroll=False)


@jax.jit
def _sc_gather(table, idx):
    kern = pl.kernel(
        _sc_gather_body,
        out_type=jax.ShapeDtypeStruct((NW_TOTAL, EMB), jnp.float32),
        mesh=plsc.VectorSubcoreMesh(core_axis_name="c", subcore_axis_name="s"),
        scratch_types=[
            pltpu.VMEM((_NCHUNK, _CHUNK), jnp.int32),
            pltpu.VMEM((_CHUNK, EMB), jnp.float32),
            pltpu.SemaphoreType.DMA,
        ],
        compiler_params=pltpu.CompilerParams(use_tc_tiling_on_sc=True),
    )
    return kern(table, idx.reshape(_NWK, _NCHUNK, _CHUNK))


_BW = 1024
_NB = _BW * WL
_GRID = NW_TOTAL // _BW
_KDIM = FWIDTH * NCHAR


def _prep_body(tbl_ref, w80_ref, out_ref):
    tbl = tbl_ref[...]
    out_ref[...] = jnp.concatenate(
        [jnp.dot(tbl, w80_ref[pl.ds(CDIM * d, CDIM), :],
                 preferred_element_type=jnp.float32)
         for d in range(FWIDTH)], axis=0)


@jax.jit
def _tc_prep(char_table, w80):
    return pl.pallas_call(
        _prep_body,
        out_shape=jax.ShapeDtypeStruct((_KDIM, FSIZE), jnp.float32),
    )(char_table, w80)


_NP = WL - FWIDTH + 1


def _conv_body(ids_ref, p_ref, b_ref, out_ref):
    iota = lax.broadcasted_iota(jnp.int32, (1, 1, NCHAR), 2)
    oh = (ids_ref[...][:, :, None] == iota).astype(jnp.bfloat16)
    oh = oh.reshape(_BW, WL * NCHAR)
    m = jnp.concatenate(
        [oh[:, NCHAR * p: NCHAR * p + _KDIM] for p in range(_NP)],
        axis=0)
    y = jnp.dot(m, p_ref[...], preferred_element_type=jnp.float32)
    y = jnp.max(y.reshape(_NP, _BW, FSIZE), axis=0)
    out_ref[...] = jnp.maximum(y + b_ref[...], 0.0)


@jax.jit
def _tc_charconv(cidx, p640, b_row):
    return pl.pallas_call(
        _conv_body,
        out_shape=jax.ShapeDtypeStruct((NW_TOTAL, FSIZE), jnp.float32),
        grid=(_GRID,),
        in_specs=[
            pl.BlockSpec((_BW, WL), lambda i: (i, 0)),
            pl.BlockSpec((_KDIM, FSIZE), lambda i: (0, 0)),
            pl.BlockSpec((1, FSIZE), lambda i: (0, 0)),
        ],
        out_specs=pl.BlockSpec((_BW, FSIZE), lambda i: (i, 0)),
    )(cidx, p640.astype(jnp.bfloat16), b_row)


def kernel(doc_w, doc_c, qry_w, qry_c, k_layer, K, W, char_table, conv_w, conv_b):
    widx = jnp.concatenate(
        [doc_w.reshape(-1), qry_w.reshape(-1)]).astype(jnp.int32)
    cidx = jnp.concatenate(
        [doc_c.reshape(-1, WL), qry_c.reshape(-1, WL)]).astype(jnp.int32)

    w80 = jnp.transpose(conv_w[:, :, 0, :], (2, 1, 0)).reshape(
        FWIDTH * CDIM, FSIZE)
    b_row = conv_b.reshape(1, FSIZE)

    p640 = _tc_prep(char_table, w80)
    w_emb = _sc_gather(W, widx)
    c_emb = _tc_charconv(cidx, p640, b_row)

    full = jnp.concatenate([w_emb, c_emb], axis=1)
    doc_emb = full[:B * DL].reshape(B, DL, EMB + FSIZE)
    qry_emb = full[B * DL:].reshape(B, QL, EMB + FSIZE)
    return doc_emb, qry_emb


# speedup vs baseline: 8.2492x; 1.0806x over previous
import functools

import jax
import jax.numpy as jnp
from jax import lax
from jax.experimental import pallas as pl
from jax.experimental.pallas import tpu as pltpu
from jax.experimental.pallas import tpu_sc as plsc

VOCAB = 100000
EMB = 128
NCHAR = 128
CDIM = 16
FSIZE = 64
FWIDTH = 5
B = 64
DL = 512
QL = 32
WL = 16

NW_TOTAL = B * DL + B * QL


_NC = 2
_NS = 16
_NWK = _NC * _NS
_PER_W = NW_TOTAL // _NWK
_NCHUNK = 17
_CHUNK = _PER_W // _NCHUNK


def _sc_gather_body(tbl_hbm, idx_hbm, out_hbm, idx_v, rows_v, sem):
    wid = lax.axis_index("s") * _NC + lax.axis_index("c")
    pltpu.sync_copy(idx_hbm.at[wid], idx_v)

    def step(c, _):
        pltpu.async_copy(tbl_hbm.at[idx_v.at[c]], rows_v, sem).wait()
        base = wid * _PER_W + c * _CHUNK
        pltpu.sync_copy(rows_v, out_hbm.at[pl.ds(base, _CHUNK)])
        return ()

    lax.fori_loop(0, _NCHUNK, step, (), unroll=False)


@jax.jit
def _sc_gather(table, idx):
    kern = pl.kernel(
        _sc_gather_body,
        out_type=jax.ShapeDtypeStruct((NW_TOTAL, EMB), jnp.float32),
        mesh=plsc.VectorSubcoreMesh(core_axis_name="c", subcore_axis_name="s"),
        scratch_types=[
            pltpu.VMEM((_NCHUNK, _CHUNK), jnp.int32),
            pltpu.VMEM((_CHUNK, EMB), jnp.float32),
            pltpu.SemaphoreType.DMA,
        ],
        compiler_params=pltpu.CompilerParams(use_tc_tiling_on_sc=True),
    )
    return kern(table, idx.reshape(_NWK, _NCHUNK, _CHUNK))


_BW = 1024
_NB = _BW * WL
_GRID = NW_TOTAL // _BW
_KDIM = FWIDTH * NCHAR


def _prep_body(tbl_ref, w80_ref, out_ref):
    tbl = tbl_ref[...]
    out_ref[...] = jnp.concatenate(
        [jnp.dot(tbl, w80_ref[pl.ds(CDIM * d, CDIM), :],
                 preferred_element_type=jnp.float32)
         for d in range(FWIDTH)], axis=0)


@jax.jit
def _tc_prep(char_table, w80):
    return pl.pallas_call(
        _prep_body,
        out_shape=jax.ShapeDtypeStruct((_KDIM, FSIZE), jnp.float32),
    )(char_table, w80)


_NP = WL - FWIDTH + 1


def _conv_body(ids_ref, p_ref, b_ref, w_ref, out_ref):
    iota = lax.broadcasted_iota(jnp.int32, (1, 1, NCHAR), 2)
    oh = (ids_ref[...][:, :, None] == iota).astype(jnp.bfloat16)
    oh = oh.reshape(_BW, WL * NCHAR)
    m = jnp.concatenate(
        [oh[:, NCHAR * p: NCHAR * p + _KDIM] for p in range(_NP)],
        axis=0)
    y = jnp.dot(m, p_ref[...], preferred_element_type=jnp.float32)
    y = jnp.max(y.reshape(_NP, _BW, FSIZE), axis=0)
    y = jnp.maximum(y + b_ref[...], 0.0)
    out_ref[...] = jnp.concatenate([w_ref[...], y], axis=1)


def _tc_charconv(cidx, p640, b_row, w_emb, n_words, step_off):
    return pl.pallas_call(
        _conv_body,
        out_shape=jax.ShapeDtypeStruct((n_words, EMB + FSIZE), jnp.float32),
        grid=(n_words // _BW,),
        in_specs=[
            pl.BlockSpec((_BW, WL), lambda i: (i + step_off, 0)),
            pl.BlockSpec((_KDIM, FSIZE), lambda i: (0, 0)),
            pl.BlockSpec((1, FSIZE), lambda i: (0, 0)),
            pl.BlockSpec((_BW, EMB), lambda i: (i + step_off, 0)),
        ],
        out_specs=pl.BlockSpec((_BW, EMB + FSIZE), lambda i: (i, 0)),
    )(cidx, p640.astype(jnp.bfloat16), b_row, w_emb)


def kernel(doc_w, doc_c, qry_w, qry_c, k_layer, K, W, char_table, conv_w, conv_b):
    widx = jnp.concatenate(
        [doc_w.reshape(-1), qry_w.reshape(-1)]).astype(jnp.int32)
    cidx = jnp.concatenate(
        [doc_c.reshape(-1, WL), qry_c.reshape(-1, WL)]).astype(jnp.int32)

    w80 = jnp.transpose(conv_w[:, :, 0, :], (2, 1, 0)).reshape(
        FWIDTH * CDIM, FSIZE)
    b_row = conv_b.reshape(1, FSIZE)

    p640 = _tc_prep(char_table, w80)
    w_emb = _sc_gather(W, widx)
    doc_full = _tc_charconv(cidx, p640, b_row, w_emb, B * DL, 0)
    qry_full = _tc_charconv(cidx, p640, b_row, w_emb, B * QL, B * DL // _BW)

    doc_emb = doc_full.reshape(B, DL, EMB + FSIZE)
    qry_emb = qry_full.reshape(B, QL, EMB + FSIZE)
    return doc_emb, qry_emb
